# Initial kernel scaffold; baseline (speedup 1.0000x reference)
import jax
import jax.numpy as jnp
from jax.experimental import pallas as pl


def kernel(feat, edge_index, feat_mask, edge_mask, W1, W2, x, y):
    raise NotImplementedError("write your pallas kernel here")


# speedup vs baseline: 20.0420x; 20.0420x over previous
import functools
from math import sqrt

import jax
import jax.numpy as jnp
from jax import lax
from jax.experimental import pallas as pl
from jax.experimental.pallas import tpu as pltpu
from jax.experimental.pallas import tpu_sc as plsc

F32 = jnp.float32
I32 = jnp.int32

NC = 2
NS = 16
NW = NC * NS
L = 16

WINT = 64
SP = 10240

_MESH = dict(core_axis_name="c", subcore_axis_name="s")
_SC_PARAMS = pltpu.CompilerParams(needs_layout_passes=False)


def _iota():
    return lax.iota(I32, L)


def _lane0():
    return _iota() == 0


def _vload(ref, base):
    return plsc.load_gather(ref, [jnp.full((L,), base, I32) + _iota()])


def _sget(ref, i):
    return plsc.load_gather(ref, [jnp.full((L,), i, I32)])[0]


def _sput(ref, i, val, enable):
    plsc.store_scatter(ref, [jnp.full((L,), i, I32)], jnp.full((L,), val),
                       mask=_lane0() & enable)


def _redsum(v):
    s = v[0]
    for j in range(1, L):
        s = s + v[j]
    return s


def _popcnt(m):
    c = plsc.all_reduce_population_count(m)
    return c if getattr(c, "ndim", 0) == 0 else c[0]


def _wid():
    return lax.axis_index("s") * NC + lax.axis_index("c")


def _sc_kernel(body, out_type, scratch_types):
    return pl.kernel(body, out_type=out_type,
                     mesh=plsc.VectorSubcoreMesh(**_MESH),
                     scratch_types=scratch_types,
                     compiler_params=_SC_PARAMS)


def _k1(n, d, ch, feat, srcp, dstp, emp, xy16,
        cnt1, src1, tag1, esc1, gate1, denxy,
        dstb, srcb, emb, clsrc, cltag, clesc, clgate,
        fxy, idxb, rows, o16, sem):
    ncw = d // L
    w = _wid()
    off = w * ch
    pltpu.sync_copy(dstp.at[pl.ds(off, ch)], dstb.at[pl.ds(0, ch)])
    pltpu.sync_copy(srcp.at[pl.ds(off, ch)], srcb.at[pl.ds(0, ch)])
    pltpu.sync_copy(emp.at[pl.ds(off, ch)], emb.at[pl.ds(0, ch)])
    pltpu.sync_copy(xy16, idxb)
    xv = idxb[...]
    x = xv[0]
    y = xv[1]
    idxb[...] = jnp.where(_iota() == 1, y, x)
    cp = pltpu.make_async_copy(feat.at[idxb], fxy, sem)
    cp.start()
    cp.wait()

    def scan(i, cnt):
        d16 = _vload(dstb, i * L)
        s16 = _vload(srcb, i * L)
        e16 = _vload(emb, i * L)
        m = (d16 == x) | (d16 == y)
        t16 = jnp.where(d16 == x, 0, 1)
        g16 = jnp.where(e16 >= 0.0, 1.0, 0.0).astype(F32)
        plsc.store_compressed(clsrc.at[pl.ds(cnt, L)], s16, mask=m)
        plsc.store_compressed(cltag.at[pl.ds(cnt, L)], t16, mask=m)
        plsc.store_compressed(clgate.at[pl.ds(cnt, L)], g16, mask=m)
        return cnt + _popcnt(m)

    cnt = lax.fori_loop(0, ch // L, scan, jnp.int32(0))

    scale = 1.0 / sqrt(float(d))

    def score_grp(k, _):
        base = k * L
        sidx = jnp.clip(_vload(clsrc, base), 0, n - 1)
        tagv = _vload(cltag, base)
        idxb[...] = sidx
        cpg = pltpu.make_async_copy(feat.at[idxb], rows, sem)
        cpg.start()
        cpg.wait()
        for j in range(L):
            accx = jnp.zeros((L,), F32)
            accy = jnp.zeros((L,), F32)
            for t in range(ncw):
                r = rows[j, pl.ds(t * L, L)]
                accx = accx + r * fxy[0, pl.ds(t * L, L)]
                accy = accy + r * fxy[1, pl.ds(t * L, L)]
            tj = tagv[j]
            sc = jnp.where(tj == 0, _redsum(accx), _redsum(accy)) * scale
            _sput(clesc, base + j, sc, (base + j) < cnt)
        return 0

    nb = (cnt + L - 1) // L
    lax.fori_loop(0, nb, score_grp, 0)

    for i in range(ch // L):
        clesc[pl.ds(i * L, L)] = jnp.exp(clesc[pl.ds(i * L, L)])

    def den_grp(k, carry):
        denx, deny = carry
        base = k * L
        escv = _vload(clesc, base)
        tagv = _vload(cltag, base)
        inl = (base + _iota()) < cnt
        ex = jnp.where(inl & (tagv == 0), escv, 0.0)
        ey = jnp.where(inl & (tagv != 0), escv, 0.0)
        return denx + _redsum(ex), deny + _redsum(ey)

    denx, deny = lax.fori_loop(0, nb, den_grp,
                               (jnp.float32(0.0), jnp.float32(0.0)))

    pltpu.sync_copy(clsrc.at[pl.ds(0, ch)], src1.at[pl.ds(off, ch)])
    pltpu.sync_copy(cltag.at[pl.ds(0, ch)], tag1.at[pl.ds(off, ch)])
    pltpu.sync_copy(clesc.at[pl.ds(0, ch)], esc1.at[pl.ds(off, ch)])
    pltpu.sync_copy(clgate.at[pl.ds(0, ch)], gate1.at[pl.ds(off, ch)])
    idxb[...] = jnp.full((L,), cnt, I32)
    pltpu.sync_copy(idxb, cnt1.at[pl.ds(w * L, L)])
    i0 = _iota()
    o16[...] = jnp.where(i0 == 0, denx, jnp.where(i0 == 1, deny, 0.0))
    pltpu.sync_copy(o16, denxy.at[pl.ds(w * L, L)])


def _k2(n, np_, ch, src1, cnt1f, nmap_hbm, u16, nmap, lsrc, cb, idxb, sem):
    w = _wid()

    @pl.when(w == 0)
    def _():
        neg = jnp.full((L,), -1, I32)

        def fill(i, _):
            nmap[pl.ds(i * L, L)] = neg
            return 0

        lax.fori_loop(0, np_ // L, fill, 0)
        pltpu.sync_copy(cnt1f, cb.at[pl.ds(0, NW * L)])

        def per_tile(t, slot):
            pltpu.sync_copy(src1.at[pl.ds(t * ch, ch)], lsrc.at[pl.ds(0, ch)])
            cnt_t = _sget(cb, t * L)

            def per_entry(i, slot):
                s = jnp.clip(_sget(lsrc, i), 0, n - 1)
                new = _sget(nmap, s) < 0
                _sput(nmap, s, slot, new)
                return slot + jnp.where(new, 1, 0)

            return lax.fori_loop(0, cnt_t, per_entry, slot)

        slot = lax.fori_loop(0, NW, per_tile, jnp.int32(0))
        pltpu.sync_copy(nmap.at[pl.ds(0, np_)], nmap_hbm)
        idxb[...] = jnp.full((L,), slot, I32)
        pltpu.sync_copy(idxb, u16)


def _k3(n, np_, d, ch, feat, srcp, dstp, emp, nmap_hbm,
        cnt2, src2, slot2, esc2, gate2, denp,
        dstb, srcb, emb, nmapb, clsrc, clslot, cldst, clesc, clgate, denb,
        idxb, idxb2, rows, rows2, sem, sem2):
    ncw = d // L
    w = _wid()
    off = w * ch
    pltpu.sync_copy(dstp.at[pl.ds(off, ch)], dstb.at[pl.ds(0, ch)])
    pltpu.sync_copy(srcp.at[pl.ds(off, ch)], srcb.at[pl.ds(0, ch)])
    pltpu.sync_copy(emp.at[pl.ds(off, ch)], emb.at[pl.ds(0, ch)])
    pltpu.sync_copy(nmap_hbm, nmapb.at[pl.ds(0, np_)])
    z = jnp.zeros((L,), F32)

    def zero(i, _):
        denb[pl.ds(i * L, L)] = z
        return 0

    lax.fori_loop(0, SP // L, zero, 0)

    def scan(i, cnt):
        d16 = _vload(dstb, i * L)
        s16 = _vload(srcb, i * L)
        e16 = _vload(emb, i * L)
        sl16 = plsc.load_gather(nmapb, [jnp.clip(d16, 0, n - 1)])
        m = (sl16 >= 0) & (d16 >= 0)
        g16 = jnp.where(e16 >= 0.0, 1.0, 0.0).astype(F32)
        plsc.store_compressed(clsrc.at[pl.ds(cnt, L)], s16, mask=m)
        plsc.store_compressed(clslot.at[pl.ds(cnt, L)], sl16, mask=m)
        plsc.store_compressed(cldst.at[pl.ds(cnt, L)], d16, mask=m)
        plsc.store_compressed(clgate.at[pl.ds(cnt, L)], g16, mask=m)
        return cnt + _popcnt(m)

    cnt = lax.fori_loop(0, ch // L, scan, jnp.int32(0))

    scale = 1.0 / sqrt(float(d))

    def score_grp(k, _):
        base = k * L
        sidx = jnp.clip(_vload(clsrc, base), 0, n - 1)
        didx = jnp.clip(_vload(cldst, base), 0, n - 1)
        idxb[...] = sidx
        idxb2[...] = didx
        cps = pltpu.make_async_copy(feat.at[idxb], rows, sem)
        cpd = pltpu.make_async_copy(feat.at[idxb2], rows2, sem2)
        cps.start()
        cpd.start()
        cps.wait()
        cpd.wait()
        for j in range(L):
            acc = jnp.zeros((L,), F32)
            for t in range(ncw):
                acc = acc + rows[j, pl.ds(t * L, L)] * rows2[j, pl.ds(t * L, L)]
            _sput(clesc, base + j, _redsum(acc) * scale, (base + j) < cnt)
        return 0

    nb = (cnt + L - 1) // L
    lax.fori_loop(0, nb, score_grp, 0)

    for i in range(ch // L):
        clesc[pl.ds(i * L, L)] = jnp.exp(clesc[pl.ds(i * L, L)])

    def den_grp(k, _):
        base = k * L
        escv = _vload(clesc, base)
        slv = jnp.clip(_vload(clslot, base), 0, SP - 1)
        esm = jnp.where((base + _iota()) < cnt, escv, 0.0)
        for j in range(L):
            plsc.addupdate_scatter(
                denb, [jnp.full((L,), slv[j], I32)],
                jnp.full((L,), esm[j]), mask=_lane0())
        return 0

    lax.fori_loop(0, nb, den_grp, 0)

    pltpu.sync_copy(clsrc.at[pl.ds(0, ch)], src2.at[pl.ds(off, ch)])
    pltpu.sync_copy(clslot.at[pl.ds(0, ch)], slot2.at[pl.ds(off, ch)])
    pltpu.sync_copy(clesc.at[pl.ds(0, ch)], esc2.at[pl.ds(off, ch)])
    pltpu.sync_copy(clgate.at[pl.ds(0, ch)], gate2.at[pl.ds(off, ch)])
    idxb[...] = jnp.full((L,), cnt, I32)
    pltpu.sync_copy(idxb, cnt2.at[pl.ds(w * L, L)])
    pltpu.sync_copy(denb.at[pl.ds(0, SP)], denp.at[pl.ds(w * SP, SP)])


def _k4(denp, den, accv, tmpv, sem):
    w = _wid()
    stripe = SP // NW
    off = w * stripe
    z = jnp.zeros((L,), F32)
    for i in range(stripe // L):
        accv[pl.ds(i * L, L)] = z

    def per_tile(t, _):
        pltpu.sync_copy(denp.at[pl.ds(t * SP + off, stripe)], tmpv)
        for i in range(stripe // L):
            accv[pl.ds(i * L, L)] = accv[pl.ds(i * L, L)] + tmpv[pl.ds(i * L, L)]
        return 0

    lax.fori_loop(0, NW, per_tile, 0)
    pltpu.sync_copy(accv, den.at[pl.ds(off, stripe)])


def _k5(n, d, feat, zeros_hbm, cnt2f, src2, slot2, esc2, gate2, den, u16,
        accp,
        denb, clsrc, clslot, clesc, clgate, cb, accb, tmps, acc4,
        rows, idxb, sidxb, spacc, sem, sem2):
    ncw = d // L
    d2 = 2 * d
    cid = lax.axis_index("c")
    sid = lax.axis_index("s")
    w = sid * NC + cid
    rpt = WINT // NS

    off = w * (clsrc.shape[0] - L)
    ch = clsrc.shape[0] - L
    pltpu.sync_copy(den, denb.at[pl.ds(0, SP)])
    pltpu.sync_copy(src2.at[pl.ds(off, ch)], clsrc.at[pl.ds(0, ch)])
    pltpu.sync_copy(slot2.at[pl.ds(off, ch)], clslot.at[pl.ds(0, ch)])
    pltpu.sync_copy(esc2.at[pl.ds(off, ch)], clesc.at[pl.ds(0, ch)])
    pltpu.sync_copy(gate2.at[pl.ds(off, ch)], clgate.at[pl.ds(0, ch)])
    pltpu.sync_copy(cnt2f, cb.at[pl.ds(0, NW * L)])
    pltpu.sync_copy(u16, idxb)
    u = idxb[...][0]
    cnt = _sget(cb, w * L)
    nwin = (u + WINT - 1) // WINT
    nb = (cnt + L - 1) // L

    def win(wi, _):
        base = wi * WINT
        pltpu.sync_copy(zeros_hbm, accb)

        def grp(k, _):
            b16 = k * L
            slv = _vload(clslot, b16)
            escv = _vload(clesc, b16)
            gatev = _vload(clgate, b16)
            srcv = jnp.clip(_vload(clsrc, b16), 0, n - 1)
            inl = (b16 + _iota()) < cnt
            inwin = inl & (slv >= base) & (slv < base + WINT)
            idxb[...] = srcv
            cpg = pltpu.make_async_copy(feat.at[idxb], rows, sem)
            cpg.start()
            cpg.wait()
            dv = plsc.load_gather(denb, [jnp.clip(slv, 0, SP - 1)])
            w0 = jnp.where(inwin, escv / (dv + 1e-15), 0.0)
            w1 = w0 * gatev
            rloc = jnp.where(inwin, slv - base, 0)
            for j in range(L):
                w0j = w0[j]
                w1j = w1[j]
                o = rloc[j] * d2
                for t in range(ncw):
                    r = rows[j, pl.ds(t * L, L)]
                    plsc.addupdate(accb.at[pl.ds(o + t * L, L)], r * w0j)
                    plsc.addupdate(accb.at[pl.ds(o + d + t * L, L)], r * w1j)
            return 0

        lax.fori_loop(0, nb, grp, 0)
        pltpu.sync_copy(accb, spacc.at[sid])
        plsc.subcore_barrier()

        r0 = sid * rpt * d2

        def redp(p, _):
            pltpu.sync_copy(spacc.at[p, pl.ds(r0, rpt * d2)], tmps)
            for i in range(rpt * d2 // L):
                acc4[pl.ds(i * L, L)] = (acc4[pl.ds(i * L, L)]
                                         + tmps[pl.ds(i * L, L)])
            return 0

        z = jnp.zeros((L,), F32)
        for i in range(rpt * d2 // L):
            acc4[pl.ds(i * L, L)] = z
        lax.fori_loop(0, NS, redp, 0)

        for r in range(rpt):
            grow = base + sid * rpt + r

            @pl.when(grow < u)
            def _():
                pltpu.sync_copy(acc4.at[pl.ds(r * d2, d2)],
                                accp.at[pl.ds(cid * (SP * d2) + grow * d2, d2)])

        plsc.subcore_barrier()
        return 0

    lax.fori_loop(0, nwin, win, 0)


def _k6(d, acc_any, w1_ref, fm_ref, u_ref, h0_any, h1_any,
        abuf, bbuf, obuf0, obuf1, sem1, sem2, sem3, sem4):
    u = u_ref[0, 0]
    bm = 128
    sigfm = jax.nn.sigmoid(fm_ref[...])

    def blk(i, _):
        r0 = i * bm
        cpa = pltpu.make_async_copy(acc_any.at[0, pl.ds(r0, bm)], abuf, sem1)
        cpb = pltpu.make_async_copy(acc_any.at[1, pl.ds(r0, bm)], bbuf, sem2)
        cpa.start()
        cpb.start()
        cpa.wait()
        cpb.wait()
        acc = abuf[...] + bbuf[...]
        a0 = acc[:, :d]
        a1 = acc[:, d:] * sigfm
        w1 = w1_ref[...]
        obuf0[...] = jnp.maximum(jnp.dot(a0, w1, preferred_element_type=F32), 0.0)
        obuf1[...] = jnp.maximum(jnp.dot(a1, w1, preferred_element_type=F32), 0.0)
        cpo0 = pltpu.make_async_copy(obuf0, h0_any.at[pl.ds(r0, bm)], sem3)
        cpo1 = pltpu.make_async_copy(obuf1, h1_any.at[pl.ds(r0, bm)], sem4)
        cpo0.start()
        cpo1.start()
        cpo0.wait()
        cpo1.wait()
        return 0

    lax.fori_loop(0, (u + bm - 1) // bm, blk, 0)


def _k7(n, np_, d, h0_hbm, h1_hbm, cnt1f, src1, tag1, esc1, gate1, denxyf,
        nmap_hbm, bpart,
        nmapb, clsrc, cltag, clesc, clgate, cb, dxyb, bacc, rows0, rows1,
        idxb, sem, sem2):
    ncw = d // L
    w = _wid()
    ch = clsrc.shape[0] - L
    off = w * ch
    pltpu.sync_copy(nmap_hbm, nmapb.at[pl.ds(0, np_)])
    pltpu.sync_copy(src1.at[pl.ds(off, ch)], clsrc.at[pl.ds(0, ch)])
    pltpu.sync_copy(tag1.at[pl.ds(off, ch)], cltag.at[pl.ds(0, ch)])
    pltpu.sync_copy(esc1.at[pl.ds(off, ch)], clesc.at[pl.ds(0, ch)])
    pltpu.sync_copy(gate1.at[pl.ds(off, ch)], clgate.at[pl.ds(0, ch)])
    pltpu.sync_copy(cnt1f, cb.at[pl.ds(0, NW * L)])
    pltpu.sync_copy(denxyf, dxyb.at[pl.ds(0, NW * L)])
    cnt = _sget(cb, w * L)

    v = jnp.zeros((L,), F32)
    for t in range(NW):
        v = v + dxyb[pl.ds(t * L, L)]
    denx = v[0]
    deny = v[1]

    z = jnp.zeros((L,), F32)
    for i in range(4 * d // L):
        bacc[pl.ds(i * L, L)] = z

    def grp(k, _):
        b16 = k * L
        srcv = jnp.clip(_vload(clsrc, b16), 0, n - 1)
        tagv = _vload(cltag, b16)
        escv = _vload(clesc, b16)
        gatev = _vload(clgate, b16)
        inl = (b16 + _iota()) < cnt
        slv = plsc.load_gather(nmapb, [srcv])
        idxb[...] = jnp.where(inl, jnp.clip(slv, 0, SP - 1), 0)
        cp0 = pltpu.make_async_copy(h0_hbm.at[idxb], rows0, sem)
        cp1 = pltpu.make_async_copy(h1_hbm.at[idxb], rows1, sem2)
        cp0.start()
        cp1.start()
        cp0.wait()
        cp1.wait()
        denl = jnp.where(tagv == 0, denx, deny)
        w0 = jnp.where(inl, escv / (denl + 1e-15), 0.0)
        w1 = w0 * gatev
        for j in range(L):
            tj = tagv[j]
            w0j = w0[j]
            w1j = w1[j]
            o0 = jnp.clip(tj, 0, 1) * d
            for t in range(ncw):
                plsc.addupdate(bacc.at[pl.ds(o0 + t * L, L)],
                               rows0[j, pl.ds(t * L, L)] * w0j)
                plsc.addupdate(bacc.at[pl.ds(2 * d + o0 + t * L, L)],
                               rows1[j, pl.ds(t * L, L)] * w1j)
        return 0

    lax.fori_loop(0, (cnt + L - 1) // L, grp, 0)
    pltpu.sync_copy(bacc.at[pl.ds(0, 4 * d)], bpart.at[pl.ds(w * 4 * d, 4 * d)])


def _k8(e, d, bp_ref, w2_ref, em_ref, fm_ref, xy_ref, out_ref):
    b = jnp.sum(bp_ref[...], axis=0)
    logit = jnp.dot(b, w2_ref[...], preferred_element_type=F32)
    neq = jnp.minimum(jnp.abs(xy_ref[0, 0] - xy_ref[0, 1]), 1).astype(F32)
    l0x = logit[0]
    l0y = logit[1] * neq + logit[0] * (1.0 - neq)
    l1x = logit[2]
    l1y = logit[3] * neq + logit[2] * (1.0 - neq)
    pred = jnp.sum(l0x * l0y)
    lp = jnp.sum(l1x * l1y)

    eps = 1e-15
    em = jax.nn.sigmoid(em_ref[...])
    s_em = jnp.sum(em)
    ent = jnp.sum(-em * jnp.log(em + eps) - (1.0 - em) * jnp.log(1.0 - em + eps))
    fm = jax.nn.sigmoid(fm_ref[...])
    m_fm = jnp.sum(fm) / float(d)
    ent2 = jnp.sum(-fm * jnp.log(fm + eps)
                   - (1.0 - fm) * jnp.log(1.0 - fm + eps)) / float(d)

    loss = (lp - pred) + 0.005 * s_em + ent / float(e) + 1.0 * m_fm + 0.1 * ent2
    out_ref[...] = jnp.reshape(loss, (1, 1))


def kernel(feat, edge_index, feat_mask, edge_mask, W1, W2, x, y):
    n, d = feat.shape
    e = edge_mask.shape[0]
    ch = -((-e) // (NW * 128)) * 128
    ep = NW * ch
    pad = ep - e
    np_ = -((-n) // L) * L
    chp = ch + L
    d2 = 2 * d

    src = edge_index[0]
    dst = edge_index[1]
    srcp = jnp.concatenate([src, jnp.zeros((pad,), I32)])
    dstp = jnp.concatenate([dst, jnp.full((pad,), -1, I32)])
    emp = jnp.concatenate([edge_mask, jnp.full((pad,), -1.0, F32)])
    xi = jnp.asarray(x, I32)
    yi = jnp.asarray(y, I32)
    xy16 = jnp.where(lax.iota(I32, L) == 1, yi, xi)
    zeros_hbm = jnp.zeros((WINT * d2,), F32)

    f32s = jax.ShapeDtypeStruct
    sdma = pltpu.SemaphoreType.DMA

    cnt1, src1, tag1, esc1, gate1, denxy = _sc_kernel(
        functools.partial(_k1, n, d, ch),
        out_type=[
            f32s((NW * L,), I32), f32s((NW * ch,), I32), f32s((NW * ch,), I32),
            f32s((NW * ch,), F32), f32s((NW * ch,), F32), f32s((NW * L,), F32),
        ],
        scratch_types=[
            pltpu.VMEM((chp,), I32), pltpu.VMEM((chp,), I32),
            pltpu.VMEM((chp,), F32),
            pltpu.VMEM((chp,), I32), pltpu.VMEM((chp,), I32),
            pltpu.VMEM((chp,), F32), pltpu.VMEM((chp,), F32),
            pltpu.VMEM((L, d), F32), pltpu.VMEM((L,), I32),
            pltpu.VMEM((L, d), F32), pltpu.VMEM((L,), F32),
            sdma,
        ],
    )(feat, srcp, dstp, emp, xy16)

    cnt1f = cnt1

    nmap_hbm, u16 = _sc_kernel(
        functools.partial(_k2, n, np_, ch),
        out_type=[f32s((np_,), I32), f32s((L,), I32)],
        scratch_types=[
            pltpu.VMEM((np_ + L,), I32), pltpu.VMEM((chp,), I32),
            pltpu.VMEM((NW * L + L,), I32), pltpu.VMEM((L,), I32), sdma,
        ],
    )(src1, cnt1f)

    cnt2, src2, slot2, esc2, gate2, denp = _sc_kernel(
        functools.partial(_k3, n, np_, d, ch),
        out_type=[
            f32s((NW * L,), I32), f32s((NW * ch,), I32), f32s((NW * ch,), I32),
            f32s((NW * ch,), F32), f32s((NW * ch,), F32), f32s((NW * SP,), F32),
        ],
        scratch_types=[
            pltpu.VMEM((chp,), I32), pltpu.VMEM((chp,), I32),
            pltpu.VMEM((chp,), F32), pltpu.VMEM((np_ + L,), I32),
            pltpu.VMEM((chp,), I32), pltpu.VMEM((chp,), I32),
            pltpu.VMEM((chp,), I32), pltpu.VMEM((chp,), F32),
            pltpu.VMEM((chp,), F32), pltpu.VMEM((SP + L,), F32),
            pltpu.VMEM((L,), I32), pltpu.VMEM((L,), I32),
            pltpu.VMEM((L, d), F32), pltpu.VMEM((L, d), F32),
            sdma, sdma,
        ],
    )(feat, srcp, dstp, emp, nmap_hbm)

    cnt2f = cnt2

    (den,) = _sc_kernel(
        _k4,
        out_type=[f32s((SP,), F32)],
        scratch_types=[
            pltpu.VMEM((SP // NW,), F32), pltpu.VMEM((SP // NW,), F32), sdma,
        ],
    )(denp)

    rpt = WINT // NS
    (accp,) = _sc_kernel(
        functools.partial(_k5, n, d),
        out_type=[f32s((NC * SP * d2,), F32)],
        scratch_types=[
            pltpu.VMEM((SP + L,), F32),
            pltpu.VMEM((chp,), I32), pltpu.VMEM((chp,), I32),
            pltpu.VMEM((chp,), F32), pltpu.VMEM((chp,), F32),
            pltpu.VMEM((NW * L + L,), I32),
            pltpu.VMEM((WINT * d2,), F32), pltpu.VMEM((rpt * d2,), F32),
            pltpu.VMEM((rpt * d2,), F32),
            pltpu.VMEM((L, d), F32), pltpu.VMEM((L,), I32),
            pltpu.VMEM((L,), I32),
            pltpu.VMEM_SHARED((NS, WINT * d2), F32),
            sdma, sdma,
        ],
    )(feat, zeros_hbm, cnt2f, src2, slot2, esc2, gate2, den, u16)

    u2d = u16[:1].reshape(1, 1)
    acc3d = accp.reshape(NC, SP, d2)
    h0, h1 = pl.pallas_call(
        functools.partial(_k6, d),
        out_shape=[f32s((SP, d), F32), f32s((SP, d), F32)],
        in_specs=[
            pl.BlockSpec(memory_space=pltpu.MemorySpace.HBM),
            pl.BlockSpec(memory_space=pltpu.MemorySpace.VMEM),
            pl.BlockSpec(memory_space=pltpu.MemorySpace.VMEM),
            pl.BlockSpec(memory_space=pltpu.MemorySpace.SMEM),
        ],
        out_specs=[pl.BlockSpec(memory_space=pltpu.MemorySpace.HBM)] * 2,
        scratch_shapes=[
            pltpu.VMEM((128, d2), F32), pltpu.VMEM((128, d2), F32),
            pltpu.VMEM((128, d), F32), pltpu.VMEM((128, d), F32),
            sdma, sdma, sdma, sdma,
        ],
    )(acc3d, W1, feat_mask, u2d)

    denxyf = denxy
    (bpart,) = _sc_kernel(
        functools.partial(_k7, n, np_, d),
        out_type=[f32s((NW * 4 * d,), F32)],
        scratch_types=[
            pltpu.VMEM((np_ + L,), I32),
            pltpu.VMEM((chp,), I32), pltpu.VMEM((chp,), I32),
            pltpu.VMEM((chp,), F32), pltpu.VMEM((chp,), F32),
            pltpu.VMEM((NW * L + L,), I32), pltpu.VMEM((NW * L + L,), F32),
            pltpu.VMEM((4 * d,), F32),
            pltpu.VMEM((L, d), F32), pltpu.VMEM((L, d), F32),
            pltpu.VMEM((L,), I32),
            sdma, sdma,
        ],
    )(h0, h1, cnt1f, src1, tag1, esc1, gate1, denxyf, nmap_hbm)

    em2d = edge_mask.reshape(e // 128, 128)
    xy2d = jnp.stack([xi, yi]).reshape(1, 2)
    bp3d = bpart.reshape(NW, 4, d)
    out = pl.pallas_call(
        functools.partial(_k8, e, d),
        out_shape=f32s((1, 1), F32),
        in_specs=[
            pl.BlockSpec(memory_space=pltpu.MemorySpace.VMEM),
            pl.BlockSpec(memory_space=pltpu.MemorySpace.VMEM),
            pl.BlockSpec(memory_space=pltpu.MemorySpace.VMEM),
            pl.BlockSpec(memory_space=pltpu.MemorySpace.VMEM),
            pl.BlockSpec(memory_space=pltpu.MemorySpace.SMEM),
        ],
    )(bp3d, W2, em2d, feat_mask, xy2d)

    return out[0, 0]


# speedup vs baseline: 24.0495x; 1.2000x over previous
import functools
from math import sqrt

import jax
import jax.numpy as jnp
from jax import lax
from jax.experimental import pallas as pl
from jax.experimental.pallas import tpu as pltpu
from jax.experimental.pallas import tpu_sc as plsc

F32 = jnp.float32
I32 = jnp.int32

NC = 2
NS = 16
NW = NC * NS
L = 16

WINT = 32
SP = 10240

_MESH = dict(core_axis_name="c", subcore_axis_name="s")
_SC_PARAMS = pltpu.CompilerParams(needs_layout_passes=False)


def _iota():
    return lax.iota(I32, L)


def _lane0():
    return _iota() == 0


def _vload(ref, base):
    return plsc.load_gather(ref, [jnp.full((L,), base, I32) + _iota()])


def _sget(ref, i):
    return plsc.load_gather(ref, [jnp.full((L,), i, I32)])[0]


def _sput(ref, i, val, enable):
    plsc.store_scatter(ref, [jnp.full((L,), i, I32)], jnp.full((L,), val),
                       mask=_lane0() & enable)


def _redsum(v):
    s = v[0]
    for j in range(1, L):
        s = s + v[j]
    return s


def _popcnt(m):
    c = plsc.all_reduce_population_count(m)
    return c if getattr(c, "ndim", 0) == 0 else c[0]


def _wid():
    return lax.axis_index("s") * NC + lax.axis_index("c")


def _sc_kernel(body, out_type, scratch_types):
    return pl.kernel(body, out_type=out_type,
                     mesh=plsc.VectorSubcoreMesh(**_MESH),
                     scratch_types=scratch_types,
                     compiler_params=_SC_PARAMS)


def _k1(n, d, ch, feat, srcp, dstp, emp, xy16,
        cnt1, src1, tag1, esc1, gate1, denxy,
        dstb, srcb, emb, clsrc, cltag, clesc, clgate,
        fxy, idxb, rows, o16, sem):
    ncw = d // L
    w = _wid()
    off = w * ch
    pltpu.sync_copy(dstp.at[pl.ds(off, ch)], dstb.at[pl.ds(0, ch)])
    pltpu.sync_copy(srcp.at[pl.ds(off, ch)], srcb.at[pl.ds(0, ch)])
    pltpu.sync_copy(emp.at[pl.ds(off, ch)], emb.at[pl.ds(0, ch)])
    pltpu.sync_copy(xy16, idxb)
    xv = idxb[...]
    x = xv[0]
    y = xv[1]
    idxb[...] = jnp.where(_iota() == 1, y, x)
    cp = pltpu.make_async_copy(feat.at[idxb], fxy, sem)
    cp.start()
    cp.wait()

    def scan(i, cnt):
        d16 = _vload(dstb, i * L)
        s16 = _vload(srcb, i * L)
        e16 = _vload(emb, i * L)
        m = (d16 == x) | (d16 == y)
        t16 = jnp.where(d16 == x, 0, 1)
        g16 = jnp.where(e16 >= 0.0, 1.0, 0.0).astype(F32)
        plsc.store_compressed(clsrc.at[pl.ds(cnt, L)], s16, mask=m)
        plsc.store_compressed(cltag.at[pl.ds(cnt, L)], t16, mask=m)
        plsc.store_compressed(clgate.at[pl.ds(cnt, L)], g16, mask=m)
        return cnt + _popcnt(m)

    cnt = lax.fori_loop(0, ch // L, scan, jnp.int32(0))

    scale = 1.0 / sqrt(float(d))

    def score_grp(k, _):
        base = k * L
        sidx = jnp.clip(_vload(clsrc, base), 0, n - 1)
        tagv = _vload(cltag, base)
        idxb[...] = sidx
        cpg = pltpu.make_async_copy(feat.at[idxb], rows, sem)
        cpg.start()
        cpg.wait()
        for j in range(L):
            accx = jnp.zeros((L,), F32)
            accy = jnp.zeros((L,), F32)
            for t in range(ncw):
                r = rows[j, pl.ds(t * L, L)]
                accx = accx + r * fxy[0, pl.ds(t * L, L)]
                accy = accy + r * fxy[1, pl.ds(t * L, L)]
            tj = tagv[j]
            sc = jnp.where(tj == 0, _redsum(accx), _redsum(accy)) * scale
            _sput(clesc, base + j, sc, (base + j) < cnt)
        return 0

    nb = (cnt + L - 1) // L
    lax.fori_loop(0, nb, score_grp, 0)

    for i in range(ch // L):
        clesc[pl.ds(i * L, L)] = jnp.exp(clesc[pl.ds(i * L, L)])

    def den_grp(k, carry):
        denx, deny = carry
        base = k * L
        escv = _vload(clesc, base)
        tagv = _vload(cltag, base)
        inl = (base + _iota()) < cnt
        ex = jnp.where(inl & (tagv == 0), escv, 0.0)
        ey = jnp.where(inl & (tagv != 0), escv, 0.0)
        return denx + _redsum(ex), deny + _redsum(ey)

    denx, deny = lax.fori_loop(0, nb, den_grp,
                               (jnp.float32(0.0), jnp.float32(0.0)))

    pltpu.sync_copy(clsrc.at[pl.ds(0, ch)], src1.at[pl.ds(off, ch)])
    pltpu.sync_copy(cltag.at[pl.ds(0, ch)], tag1.at[pl.ds(off, ch)])
    pltpu.sync_copy(clesc.at[pl.ds(0, ch)], esc1.at[pl.ds(off, ch)])
    pltpu.sync_copy(clgate.at[pl.ds(0, ch)], gate1.at[pl.ds(off, ch)])
    idxb[...] = jnp.full((L,), cnt, I32)
    pltpu.sync_copy(idxb, cnt1.at[pl.ds(w * L, L)])
    i0 = _iota()
    o16[...] = jnp.where(i0 == 0, denx, jnp.where(i0 == 1, deny, 0.0))
    pltpu.sync_copy(o16, denxy.at[pl.ds(w * L, L)])


def _k2(n, np_, ch, src1, cnt1f, nmap_hbm, u16, nmap, lsrc, cb, idxb, sem):
    w = _wid()

    @pl.when(w == 0)
    def _():
        neg = jnp.full((L,), -1, I32)

        def fill(i, _):
            nmap[pl.ds(i * L, L)] = neg
            return 0

        lax.fori_loop(0, np_ // L, fill, 0)
        pltpu.sync_copy(cnt1f, cb.at[pl.ds(0, NW * L)])

        def per_tile(t, slot):
            pltpu.sync_copy(src1.at[pl.ds(t * ch, ch)], lsrc.at[pl.ds(0, ch)])
            cnt_t = _sget(cb, t * L)

            def per_entry(i, slot):
                s = jnp.clip(_sget(lsrc, i), 0, n - 1)
                new = _sget(nmap, s) < 0
                _sput(nmap, s, slot, new)
                return slot + jnp.where(new, 1, 0)

            return lax.fori_loop(0, cnt_t, per_entry, slot)

        slot = lax.fori_loop(0, NW, per_tile, jnp.int32(0))
        pltpu.sync_copy(nmap.at[pl.ds(0, np_)], nmap_hbm)
        idxb[...] = jnp.full((L,), slot, I32)
        pltpu.sync_copy(idxb, u16)


def _kb(n, np_, d, ch, nsub, feat, srcp, dstp, emp, nmap_hbm, u16, zeros_hbm,
        accp,
        dstb, srcb, emb, nmapb, clsrc, clslot, cldst, clesc, clgate, denb,
        rows, rows2, accb, tmps, acc4, dstripe, tmpd, idxb, idxb2,
        spacc, sem, sem2):
    ncw = d // L
    d2 = 2 * d
    subc = ch // nsub
    cid = lax.axis_index("c")
    sid = lax.axis_index("s")
    rpt = WINT // NS

    pltpu.sync_copy(nmap_hbm, nmapb.at[pl.ds(0, np_)])
    pltpu.sync_copy(u16, idxb)
    u = idxb[...][0]
    z = jnp.zeros((L,), F32)

    def zden(i, _):
        denb[pl.ds(i * L, L)] = z
        return 0

    lax.fori_loop(0, SP // L, zden, 0)

    cnt = jnp.int32(0)
    for sub in range(nsub):
        soff = sid * ch + sub * subc
        pltpu.sync_copy(dstp.at[pl.ds(soff, subc)], dstb.at[pl.ds(0, subc)])
        pltpu.sync_copy(srcp.at[pl.ds(soff, subc)], srcb.at[pl.ds(0, subc)])
        pltpu.sync_copy(emp.at[pl.ds(soff, subc)], emb.at[pl.ds(0, subc)])

        def scan(i, cnt):
            d16 = _vload(dstb, i * L)
            s16 = _vload(srcb, i * L)
            e16 = _vload(emb, i * L)
            sl16 = plsc.load_gather(nmapb, [jnp.clip(d16, 0, n - 1)])
            m = (sl16 >= 0) & (d16 >= 0)
            g16 = jnp.where(e16 >= 0.0, 1.0, 0.0).astype(F32)
            plsc.store_compressed(clsrc.at[pl.ds(cnt, L)], s16, mask=m)
            plsc.store_compressed(clslot.at[pl.ds(cnt, L)], sl16, mask=m)
            plsc.store_compressed(cldst.at[pl.ds(cnt, L)], d16, mask=m)
            plsc.store_compressed(clgate.at[pl.ds(cnt, L)], g16, mask=m)
            return cnt + _popcnt(m)

        cnt = lax.fori_loop(0, subc // L, scan, cnt)

    scale = 1.0 / sqrt(float(d))

    def score_grp(k, _):
        base = k * L
        sidx = jnp.clip(_vload(clsrc, base), 0, n - 1)
        didx = jnp.clip(_vload(cldst, base), 0, n - 1)
        idxb[...] = sidx
        idxb2[...] = didx
        cps = pltpu.make_async_copy(feat.at[idxb], rows, sem)
        cpd = pltpu.make_async_copy(feat.at[idxb2], rows2, sem2)
        cps.start()
        cpd.start()
        cps.wait()
        cpd.wait()
        for j in range(L):
            acc = jnp.zeros((L,), F32)
            for t in range(ncw):
                acc = acc + rows[j, pl.ds(t * L, L)] * rows2[j, pl.ds(t * L, L)]
            _sput(clesc, base + j, _redsum(acc) * scale, (base + j) < cnt)
        return 0

    nb = (cnt + L - 1) // L
    lax.fori_loop(0, nb, score_grp, 0)

    for i in range(ch // L):
        clesc[pl.ds(i * L, L)] = jnp.exp(clesc[pl.ds(i * L, L)])

    def den_grp(k, _):
        base = k * L
        escv = _vload(clesc, base)
        slv = jnp.clip(_vload(clslot, base), 0, SP - 1)
        esm = jnp.where((base + _iota()) < cnt, escv, 0.0)
        for j in range(L):
            plsc.addupdate_scatter(
                denb, [jnp.full((L,), slv[j], I32)],
                jnp.full((L,), esm[j]), mask=_lane0())
        return 0

    lax.fori_loop(0, nb, den_grp, 0)

    stripe = SP // NS
    pltpu.sync_copy(denb.at[pl.ds(0, SP)], spacc.at[sid, pl.ds(0, SP)])
    plsc.subcore_barrier()
    soff2 = sid * stripe
    for i in range(stripe // L):
        dstripe[pl.ds(i * L, L)] = z

    def dred(t, _):
        pltpu.sync_copy(spacc.at[t, pl.ds(soff2, stripe)],
                        tmpd.at[pl.ds(0, stripe)])
        for i in range(stripe // L):
            dstripe[pl.ds(i * L, L)] = (dstripe[pl.ds(i * L, L)]
                                        + tmpd[pl.ds(i * L, L)])
        return 0

    lax.fori_loop(0, NS, dred, 0)
    pltpu.sync_copy(dstripe.at[pl.ds(0, stripe)],
                    spacc.at[sid, pl.ds(SP, stripe)])
    plsc.subcore_barrier()

    def dget(t, _):
        pltpu.sync_copy(spacc.at[t, pl.ds(SP, stripe)],
                        denb.at[pl.ds(t * stripe, stripe)])
        return 0

    lax.fori_loop(0, NS, dget, 0)
    plsc.subcore_barrier()

    nwin = (u + WINT - 1) // WINT
    nwin_me = jnp.maximum((nwin - cid + 1) // 2, 0)

    def win(wi2, _):
        base = (wi2 * 2 + cid) * WINT
        pltpu.sync_copy(zeros_hbm, accb)

        def grp(k, _):
            b16 = k * L
            slv = _vload(clslot, b16)
            escv = _vload(clesc, b16)
            gatev = _vload(clgate, b16)
            srcv = jnp.clip(_vload(clsrc, b16), 0, n - 1)
            inl = (b16 + _iota()) < cnt
            inwin = inl & (slv >= base) & (slv < base + WINT)
            idxb[...] = srcv
            cpg = pltpu.make_async_copy(feat.at[idxb], rows, sem)
            cpg.start()
            cpg.wait()
            dv = plsc.load_gather(denb, [jnp.clip(slv, 0, SP - 1)])
            w0 = jnp.where(inwin, escv / (dv + 1e-15), 0.0)
            w1 = w0 * gatev
            rloc = jnp.where(inwin, slv - base, 0)
            for j in range(L):
                w0j = w0[j]
                w1j = w1[j]
                o = rloc[j] * d2
                for t in range(ncw):
                    r = rows[j, pl.ds(t * L, L)]
                    plsc.addupdate(accb.at[pl.ds(o + t * L, L)], r * w0j)
                    plsc.addupdate(accb.at[pl.ds(o + d + t * L, L)], r * w1j)
            return 0

        lax.fori_loop(0, nb, grp, 0)
        pltpu.sync_copy(accb, spacc.at[sid])
        plsc.subcore_barrier()

        r0 = sid * rpt * d2
        for i in range(rpt * d2 // L):
            acc4[pl.ds(i * L, L)] = z

        def redp(pt, _):
            pltpu.sync_copy(spacc.at[pt, pl.ds(r0, rpt * d2)], tmps)
            for i in range(rpt * d2 // L):
                acc4[pl.ds(i * L, L)] = (acc4[pl.ds(i * L, L)]
                                         + tmps[pl.ds(i * L, L)])
            return 0

        lax.fori_loop(0, NS, redp, 0)

        for r in range(rpt):
            grow = base + sid * rpt + r

            @pl.when(grow < u)
            def _():
                pltpu.sync_copy(acc4.at[pl.ds(r * d2, d2)],
                                accp.at[pl.ds(grow * d2, d2)])

        plsc.subcore_barrier()
        return 0

    lax.fori_loop(0, nwin_me, win, 0)


def _k6(d, acc_any, w1_ref, fm_ref, u_ref, h0_any, h1_any,
        abuf, obuf0, obuf1, sem1, sem3, sem4):
    u = u_ref[0, 0]
    bm = 128
    sigfm = jax.nn.sigmoid(fm_ref[...])

    def blk(i, _):
        r0 = i * bm
        cpa = pltpu.make_async_copy(acc_any.at[pl.ds(r0, bm)], abuf, sem1)
        cpa.start()
        cpa.wait()
        acc = abuf[...]
        a0 = acc[:, :d]
        a1 = acc[:, d:] * sigfm
        w1 = w1_ref[...]
        obuf0[...] = jnp.maximum(jnp.dot(a0, w1, preferred_element_type=F32), 0.0)
        obuf1[...] = jnp.maximum(jnp.dot(a1, w1, preferred_element_type=F32), 0.0)
        cpo0 = pltpu.make_async_copy(obuf0, h0_any.at[pl.ds(r0, bm)], sem3)
        cpo1 = pltpu.make_async_copy(obuf1, h1_any.at[pl.ds(r0, bm)], sem4)
        cpo0.start()
        cpo1.start()
        cpo0.wait()
        cpo1.wait()
        return 0

    lax.fori_loop(0, (u + bm - 1) // bm, blk, 0)


def _k7(n, np_, d, h0_hbm, h1_hbm, cnt1f, src1, tag1, esc1, gate1, denxyf,
        nmap_hbm, bpart,
        nmapb, clsrc, cltag, clesc, clgate, cb, dxyb, bacc, rows0, rows1,
        idxb, sem, sem2):
    ncw = d // L
    w = _wid()
    ch = clsrc.shape[0] - L
    off = w * ch
    pltpu.sync_copy(nmap_hbm, nmapb.at[pl.ds(0, np_)])
    pltpu.sync_copy(src1.at[pl.ds(off, ch)], clsrc.at[pl.ds(0, ch)])
    pltpu.sync_copy(tag1.at[pl.ds(off, ch)], cltag.at[pl.ds(0, ch)])
    pltpu.sync_copy(esc1.at[pl.ds(off, ch)], clesc.at[pl.ds(0, ch)])
    pltpu.sync_copy(gate1.at[pl.ds(off, ch)], clgate.at[pl.ds(0, ch)])
    pltpu.sync_copy(cnt1f, cb.at[pl.ds(0, NW * L)])
    pltpu.sync_copy(denxyf, dxyb.at[pl.ds(0, NW * L)])
    cnt = _sget(cb, w * L)

    v = jnp.zeros((L,), F32)
    for t in range(NW):
        v = v + dxyb[pl.ds(t * L, L)]
    denx = v[0]
    deny = v[1]

    z = jnp.zeros((L,), F32)
    for i in range(4 * d // L):
        bacc[pl.ds(i * L, L)] = z

    def grp(k, _):
        b16 = k * L
        srcv = jnp.clip(_vload(clsrc, b16), 0, n - 1)
        tagv = _vload(cltag, b16)
        escv = _vload(clesc, b16)
        gatev = _vload(clgate, b16)
        inl = (b16 + _iota()) < cnt
        slv = plsc.load_gather(nmapb, [srcv])
        idxb[...] = jnp.where(inl, jnp.clip(slv, 0, SP - 1), 0)
        cp0 = pltpu.make_async_copy(h0_hbm.at[idxb], rows0, sem)
        cp1 = pltpu.make_async_copy(h1_hbm.at[idxb], rows1, sem2)
        cp0.start()
        cp1.start()
        cp0.wait()
        cp1.wait()
        denl = jnp.where(tagv == 0, denx, deny)
        w0 = jnp.where(inl, escv / (denl + 1e-15), 0.0)
        w1 = w0 * gatev
        for j in range(L):
            tj = tagv[j]
            w0j = w0[j]
            w1j = w1[j]
            o0 = jnp.clip(tj, 0, 1) * d
            for t in range(ncw):
                plsc.addupdate(bacc.at[pl.ds(o0 + t * L, L)],
                               rows0[j, pl.ds(t * L, L)] * w0j)
                plsc.addupdate(bacc.at[pl.ds(2 * d + o0 + t * L, L)],
                               rows1[j, pl.ds(t * L, L)] * w1j)
        return 0

    lax.fori_loop(0, (cnt + L - 1) // L, grp, 0)
    pltpu.sync_copy(bacc.at[pl.ds(0, 4 * d)], bpart.at[pl.ds(w * 4 * d, 4 * d)])


def _k8(e, d, bp_ref, w2_ref, em_ref, fm_ref, xy_ref, out_ref):
    b = jnp.sum(bp_ref[...], axis=0)
    logit = jnp.dot(b, w2_ref[...], preferred_element_type=F32)
    neq = jnp.minimum(jnp.abs(xy_ref[0, 0] - xy_ref[0, 1]), 1).astype(F32)
    l0x = logit[0]
    l0y = logit[1] * neq + logit[0] * (1.0 - neq)
    l1x = logit[2]
    l1y = logit[3] * neq + logit[2] * (1.0 - neq)
    pred = jnp.sum(l0x * l0y)
    lp = jnp.sum(l1x * l1y)

    eps = 1e-15
    em = jax.nn.sigmoid(em_ref[...])
    s_em = jnp.sum(em)
    ent = jnp.sum(-em * jnp.log(em + eps) - (1.0 - em) * jnp.log(1.0 - em + eps))
    fm = jax.nn.sigmoid(fm_ref[...])
    m_fm = jnp.sum(fm) / float(d)
    ent2 = jnp.sum(-fm * jnp.log(fm + eps)
                   - (1.0 - fm) * jnp.log(1.0 - fm + eps)) / float(d)

    loss = (lp - pred) + 0.005 * s_em + ent / float(e) + 1.0 * m_fm + 0.1 * ent2
    out_ref[...] = jnp.reshape(loss, (1, 1))


def kernel(feat, edge_index, feat_mask, edge_mask, W1, W2, x, y):
    n, d = feat.shape
    e = edge_mask.shape[0]
    ch = -((-e) // (NW * 128)) * 128
    ep = NW * ch
    pad = ep - e
    np_ = -((-n) // L) * L
    chp = ch + L
    d2 = 2 * d

    src = edge_index[0]
    dst = edge_index[1]
    srcp = jnp.concatenate([src, jnp.zeros((pad,), I32)])
    dstp = jnp.concatenate([dst, jnp.full((pad,), -1, I32)])
    emp = jnp.concatenate([edge_mask, jnp.full((pad,), -1.0, F32)])
    xi = jnp.asarray(x, I32)
    yi = jnp.asarray(y, I32)
    xy16 = jnp.where(lax.iota(I32, L) == 1, yi, xi)
    zeros_hbm = jnp.zeros((WINT * d2,), F32)

    f32s = jax.ShapeDtypeStruct
    sdma = pltpu.SemaphoreType.DMA

    cnt1, src1, tag1, esc1, gate1, denxy = _sc_kernel(
        functools.partial(_k1, n, d, ch),
        out_type=[
            f32s((NW * L,), I32), f32s((NW * ch,), I32), f32s((NW * ch,), I32),
            f32s((NW * ch,), F32), f32s((NW * ch,), F32), f32s((NW * L,), F32),
        ],
        scratch_types=[
            pltpu.VMEM((chp,), I32), pltpu.VMEM((chp,), I32),
            pltpu.VMEM((chp,), F32),
            pltpu.VMEM((chp,), I32), pltpu.VMEM((chp,), I32),
            pltpu.VMEM((chp,), F32), pltpu.VMEM((chp,), F32),
            pltpu.VMEM((L, d), F32), pltpu.VMEM((L,), I32),
            pltpu.VMEM((L, d), F32), pltpu.VMEM((L,), F32),
            sdma,
        ],
    )(feat, srcp, dstp, emp, xy16)

    cnt1f = cnt1

    nmap_hbm, u16 = _sc_kernel(
        functools.partial(_k2, n, np_, ch),
        out_type=[f32s((np_,), I32), f32s((L,), I32)],
        scratch_types=[
            pltpu.VMEM((np_ + L,), I32), pltpu.VMEM((chp,), I32),
            pltpu.VMEM((NW * L + L,), I32), pltpu.VMEM((L,), I32), sdma,
        ],
    )(src1, cnt1f)

    ch16 = -((-e) // (NS * 128)) * 128
    nsub = 4
    subc = ch16 // nsub
    rpt = WINT // NS
    stripe = SP // NS
    (accp,) = _sc_kernel(
        functools.partial(_kb, n, np_, d, ch16, nsub),
        out_type=[f32s((SP * d2,), F32)],
        scratch_types=[
            pltpu.VMEM((subc + L,), I32), pltpu.VMEM((subc + L,), I32),
            pltpu.VMEM((subc + L,), F32), pltpu.VMEM((np_ + L,), I32),
            pltpu.VMEM((ch16 + L,), I32), pltpu.VMEM((ch16 + L,), I32),
            pltpu.VMEM((ch16 + L,), I32), pltpu.VMEM((ch16 + L,), F32),
            pltpu.VMEM((ch16 + L,), F32), pltpu.VMEM((SP + L,), F32),
            pltpu.VMEM((L, d), F32), pltpu.VMEM((L, d), F32),
            pltpu.VMEM((WINT * d2,), F32), pltpu.VMEM((rpt * d2,), F32),
            pltpu.VMEM((rpt * d2,), F32),
            pltpu.VMEM((stripe + L,), F32), pltpu.VMEM((stripe + L,), F32),
            pltpu.VMEM((L,), I32), pltpu.VMEM((L,), I32),
            pltpu.VMEM_SHARED((NS, WINT * d2), F32),
            sdma, sdma,
        ],
    )(feat, srcp, dstp, emp, nmap_hbm, u16, zeros_hbm)

    u2d = u16[:1].reshape(1, 1)
    acc2d = accp.reshape(SP, d2)
    h0, h1 = pl.pallas_call(
        functools.partial(_k6, d),
        out_shape=[f32s((SP, d), F32), f32s((SP, d), F32)],
        in_specs=[
            pl.BlockSpec(memory_space=pltpu.MemorySpace.HBM),
            pl.BlockSpec(memory_space=pltpu.MemorySpace.VMEM),
            pl.BlockSpec(memory_space=pltpu.MemorySpace.VMEM),
            pl.BlockSpec(memory_space=pltpu.MemorySpace.SMEM),
        ],
        out_specs=[pl.BlockSpec(memory_space=pltpu.MemorySpace.HBM)] * 2,
        scratch_shapes=[
            pltpu.VMEM((128, d2), F32),
            pltpu.VMEM((128, d), F32), pltpu.VMEM((128, d), F32),
            sdma, sdma, sdma,
        ],
    )(acc2d, W1, feat_mask, u2d)

    denxyf = denxy
    (bpart,) = _sc_kernel(
        functools.partial(_k7, n, np_, d),
        out_type=[f32s((NW * 4 * d,), F32)],
        scratch_types=[
            pltpu.VMEM((np_ + L,), I32),
            pltpu.VMEM((chp,), I32), pltpu.VMEM((chp,), I32),
            pltpu.VMEM((chp,), F32), pltpu.VMEM((chp,), F32),
            pltpu.VMEM((NW * L + L,), I32), pltpu.VMEM((NW * L + L,), F32),
            pltpu.VMEM((4 * d,), F32),
            pltpu.VMEM((L, d), F32), pltpu.VMEM((L, d), F32),
            pltpu.VMEM((L,), I32),
            sdma, sdma,
        ],
    )(h0, h1, cnt1f, src1, tag1, esc1, gate1, denxyf, nmap_hbm)

    em2d = edge_mask.reshape(e // 128, 128)
    xy2d = jnp.stack([xi, yi]).reshape(1, 2)
    bp3d = bpart.reshape(NW, 4, d)
    out = pl.pallas_call(
        functools.partial(_k8, e, d),
        out_shape=f32s((1, 1), F32),
        in_specs=[
            pl.BlockSpec(memory_space=pltpu.MemorySpace.VMEM),
            pl.BlockSpec(memory_space=pltpu.MemorySpace.VMEM),
            pl.BlockSpec(memory_space=pltpu.MemorySpace.VMEM),
            pl.BlockSpec(memory_space=pltpu.MemorySpace.VMEM),
            pl.BlockSpec(memory_space=pltpu.MemorySpace.SMEM),
        ],
    )(bp3d, W2, em2d, feat_mask, xy2d)

    return out[0, 0]


# speedup vs baseline: 25.1723x; 1.0467x over previous
import functools
from math import sqrt

import jax
import jax.numpy as jnp
from jax import lax
from jax.experimental import pallas as pl
from jax.experimental.pallas import tpu as pltpu
from jax.experimental.pallas import tpu_sc as plsc

F32 = jnp.float32
I32 = jnp.int32

NC = 2
NS = 16
NW = NC * NS
L = 16

WINT = 32
SP = 10240

_MESH = dict(core_axis_name="c", subcore_axis_name="s")
_SC_PARAMS = pltpu.CompilerParams(needs_layout_passes=False)


def _iota():
    return lax.iota(I32, L)


def _lane0():
    return _iota() == 0


def _vload(ref, base):
    return plsc.load_gather(ref, [jnp.full((L,), base, I32) + _iota()])


def _sget(ref, i):
    return plsc.load_gather(ref, [jnp.full((L,), i, I32)])[0]


def _sput(ref, i, val, enable):
    plsc.store_scatter(ref, [jnp.full((L,), i, I32)], jnp.full((L,), val),
                       mask=_lane0() & enable)


def _redsum(v):
    s = v[0]
    for j in range(1, L):
        s = s + v[j]
    return s


def _popcnt(m):
    c = plsc.all_reduce_population_count(m)
    return c if getattr(c, "ndim", 0) == 0 else c[0]


def _wid():
    return lax.axis_index("s") * NC + lax.axis_index("c")


def _sc_kernel(body, out_type, scratch_types):
    return pl.kernel(body, out_type=out_type,
                     mesh=plsc.VectorSubcoreMesh(**_MESH),
                     scratch_types=scratch_types,
                     compiler_params=_SC_PARAMS)


def _ka(n, np_, d, ch, nsub, feat, srcp, dstp, emp, xy16,
        cnt1, src1, tag1, esc1, gate1, denxy, nmap_hbm, u16,
        dstb, srcb, emb, clsrc, cltag, clesc, clgate, nmapb,
        fxy, rows, idxb, o16, spl, sem):
    ncw = d // L
    subc = ch // nsub
    cid = lax.axis_index("c")
    sid = lax.axis_index("s")

    pltpu.sync_copy(xy16, idxb)
    xv = idxb[...]
    x = xv[0]
    y = xv[1]
    idxb[...] = jnp.where(_iota() == 1, y, x)
    cp = pltpu.make_async_copy(feat.at[idxb], fxy, sem)
    cp.start()
    cp.wait()

    cnt = jnp.int32(0)
    for sub in range(nsub):
        soff = sid * ch + sub * subc
        pltpu.sync_copy(dstp.at[pl.ds(soff, subc)], dstb.at[pl.ds(0, subc)])
        pltpu.sync_copy(srcp.at[pl.ds(soff, subc)], srcb.at[pl.ds(0, subc)])
        pltpu.sync_copy(emp.at[pl.ds(soff, subc)], emb.at[pl.ds(0, subc)])

        def scan(i, cnt):
            d16 = _vload(dstb, i * L)
            s16 = _vload(srcb, i * L)
            e16 = _vload(emb, i * L)
            m = (d16 == x) | (d16 == y)
            t16 = jnp.where(d16 == x, 0, 1)
            g16 = jnp.where(e16 >= 0.0, 1.0, 0.0).astype(F32)
            plsc.store_compressed(clsrc.at[pl.ds(cnt, L)], s16, mask=m)
            plsc.store_compressed(cltag.at[pl.ds(cnt, L)], t16, mask=m)
            plsc.store_compressed(clgate.at[pl.ds(cnt, L)], g16, mask=m)
            return cnt + _popcnt(m)

        cnt = lax.fori_loop(0, subc // L, scan, cnt)

    scale = 1.0 / sqrt(float(d))

    def score_grp(k, _):
        base = k * L
        sidx = jnp.clip(_vload(clsrc, base), 0, n - 1)
        tagv = _vload(cltag, base)
        idxb[...] = sidx
        cpg = pltpu.make_async_copy(feat.at[idxb], rows, sem)
        cpg.start()
        cpg.wait()
        for j in range(L):
            accx = jnp.zeros((L,), F32)
            accy = jnp.zeros((L,), F32)
            for t in range(ncw):
                r = rows[j, pl.ds(t * L, L)]
                accx = accx + r * fxy[0, pl.ds(t * L, L)]
                accy = accy + r * fxy[1, pl.ds(t * L, L)]
            tj = tagv[j]
            sc = jnp.where(tj == 0, _redsum(accx), _redsum(accy)) * scale
            _sput(clesc, base + j, sc, (base + j) < cnt)
        return 0

    nb = (cnt + L - 1) // L
    lax.fori_loop(0, nb, score_grp, 0)

    for i in range(ch // L):
        clesc[pl.ds(i * L, L)] = jnp.exp(clesc[pl.ds(i * L, L)])

    def den_grp(k, carry):
        denx, deny = carry
        base = k * L
        escv = _vload(clesc, base)
        tagv = _vload(cltag, base)
        inl = (base + _iota()) < cnt
        ex = jnp.where(inl & (tagv == 0), escv, 0.0)
        ey = jnp.where(inl & (tagv != 0), escv, 0.0)
        return denx + _redsum(ex), deny + _redsum(ey)

    denx, deny = lax.fori_loop(0, nb, den_grp,
                               (jnp.float32(0.0), jnp.float32(0.0)))

    pltpu.sync_copy(clsrc.at[pl.ds(0, ch)], spl.at[sid, pl.ds(0, ch)])
    idxb[...] = jnp.full((L,), cnt, I32)
    pltpu.sync_copy(idxb, spl.at[sid, pl.ds(ch, L)])

    @pl.when(cid == 0)
    def _():
        off = sid * ch
        pltpu.sync_copy(clsrc.at[pl.ds(0, ch)], src1.at[pl.ds(off, ch)])
        pltpu.sync_copy(cltag.at[pl.ds(0, ch)], tag1.at[pl.ds(off, ch)])
        pltpu.sync_copy(clesc.at[pl.ds(0, ch)], esc1.at[pl.ds(off, ch)])
        pltpu.sync_copy(clgate.at[pl.ds(0, ch)], gate1.at[pl.ds(off, ch)])
        pltpu.sync_copy(idxb, cnt1.at[pl.ds(sid * L, L)])
        i0 = _iota()
        o16[...] = jnp.where(i0 == 0, denx, jnp.where(i0 == 1, deny, 0.0))
        pltpu.sync_copy(o16, denxy.at[pl.ds(sid * L, L)])

    plsc.subcore_barrier()

    @pl.when(sid == 0)
    def _():
        neg = jnp.full((L,), -1, I32)

        def fill(i, _):
            nmapb[pl.ds(i * L, L)] = neg
            return 0

        lax.fori_loop(0, np_ // L, fill, 0)

        def per_tile(t, slot):
            pltpu.sync_copy(spl.at[t, pl.ds(0, ch)], clsrc.at[pl.ds(0, ch)])
            pltpu.sync_copy(spl.at[t, pl.ds(ch, L)], idxb)
            cnt_t = idxb[...][0]

            def per_entry(i, slot):
                sv = jnp.clip(_sget(clsrc, i), 0, n - 1)
                new = _sget(nmapb, sv) < 0
                _sput(nmapb, sv, slot, new)
                return slot + jnp.where(new, 1, 0)

            return lax.fori_loop(0, cnt_t, per_entry, slot)

        slot = lax.fori_loop(0, NS, per_tile, jnp.int32(0))

        @pl.when(cid == 0)
        def _():
            pltpu.sync_copy(nmapb.at[pl.ds(0, np_)], nmap_hbm)
            idxb[...] = jnp.full((L,), slot, I32)
            pltpu.sync_copy(idxb, u16)


def _kb(n, np_, d, ch, nsub, feat, srcp, dstp, emp, nmap_hbm, u16, zeros_hbm,
        accp,
        dstb, srcb, emb, nmapb, clsrc, clslot, cldst, clesc, clgate, denb,
        rows, rows2, accb, tmps, acc4, dstripe, tmpd, idxb, idxb2,
        spacc, sem, sem2):
    ncw = d // L
    d2 = 2 * d
    subc = ch // nsub
    cid = lax.axis_index("c")
    sid = lax.axis_index("s")
    rpt = WINT // NS

    pltpu.sync_copy(nmap_hbm, nmapb.at[pl.ds(0, np_)])
    pltpu.sync_copy(u16, idxb)
    u = idxb[...][0]
    z = jnp.zeros((L,), F32)

    def zden(i, _):
        denb[pl.ds(i * L, L)] = z
        return 0

    lax.fori_loop(0, SP // L, zden, 0)

    cnt = jnp.int32(0)
    for sub in range(nsub):
        soff = sid * ch + sub * subc
        pltpu.sync_copy(dstp.at[pl.ds(soff, subc)], dstb.at[pl.ds(0, subc)])
        pltpu.sync_copy(srcp.at[pl.ds(soff, subc)], srcb.at[pl.ds(0, subc)])
        pltpu.sync_copy(emp.at[pl.ds(soff, subc)], emb.at[pl.ds(0, subc)])

        def scan(i, cnt):
            d16 = _vload(dstb, i * L)
            s16 = _vload(srcb, i * L)
            e16 = _vload(emb, i * L)
            sl16 = plsc.load_gather(nmapb, [jnp.clip(d16, 0, n - 1)])
            m = (sl16 >= 0) & (d16 >= 0)
            g16 = jnp.where(e16 >= 0.0, 1.0, 0.0).astype(F32)
            plsc.store_compressed(clsrc.at[pl.ds(cnt, L)], s16, mask=m)
            plsc.store_compressed(clslot.at[pl.ds(cnt, L)], sl16, mask=m)
            plsc.store_compressed(cldst.at[pl.ds(cnt, L)], d16, mask=m)
            plsc.store_compressed(clgate.at[pl.ds(cnt, L)], g16, mask=m)
            return cnt + _popcnt(m)

        cnt = lax.fori_loop(0, subc // L, scan, cnt)

    scale = 1.0 / sqrt(float(d))

    def score_grp(k, _):
        base = k * L
        sidx = jnp.clip(_vload(clsrc, base), 0, n - 1)
        didx = jnp.clip(_vload(cldst, base), 0, n - 1)
        idxb[...] = sidx
        idxb2[...] = didx
        cps = pltpu.make_async_copy(feat.at[idxb], rows, sem)
        cpd = pltpu.make_async_copy(feat.at[idxb2], rows2, sem2)
        cps.start()
        cpd.start()
        cps.wait()
        cpd.wait()
        for j in range(L):
            acc = jnp.zeros((L,), F32)
            for t in range(ncw):
                acc = acc + rows[j, pl.ds(t * L, L)] * rows2[j, pl.ds(t * L, L)]
            _sput(clesc, base + j, _redsum(acc) * scale, (base + j) < cnt)
        return 0

    nb = (cnt + L - 1) // L
    lax.fori_loop(0, nb, score_grp, 0)

    for i in range(ch // L):
        clesc[pl.ds(i * L, L)] = jnp.exp(clesc[pl.ds(i * L, L)])

    def den_grp(k, _):
        base = k * L
        escv = _vload(clesc, base)
        slv = jnp.clip(_vload(clslot, base), 0, SP - 1)
        esm = jnp.where((base + _iota()) < cnt, escv, 0.0)
        for j in range(L):
            plsc.addupdate_scatter(
                denb, [jnp.full((L,), slv[j], I32)],
                jnp.full((L,), esm[j]), mask=_lane0())
        return 0

    lax.fori_loop(0, nb, den_grp, 0)

    stripe = SP // NS
    pltpu.sync_copy(denb.at[pl.ds(0, SP)], spacc.at[sid, pl.ds(0, SP)])
    plsc.subcore_barrier()
    soff2 = sid * stripe
    for i in range(stripe // L):
        dstripe[pl.ds(i * L, L)] = z

    def dred(t, _):
        pltpu.sync_copy(spacc.at[t, pl.ds(soff2, stripe)],
                        tmpd.at[pl.ds(0, stripe)])
        for i in range(stripe // L):
            dstripe[pl.ds(i * L, L)] = (dstripe[pl.ds(i * L, L)]
                                        + tmpd[pl.ds(i * L, L)])
        return 0

    lax.fori_loop(0, NS, dred, 0)
    pltpu.sync_copy(dstripe.at[pl.ds(0, stripe)],
                    spacc.at[sid, pl.ds(SP, stripe)])
    plsc.subcore_barrier()

    def dget(t, _):
        pltpu.sync_copy(spacc.at[t, pl.ds(SP, stripe)],
                        denb.at[pl.ds(t * stripe, stripe)])
        return 0

    lax.fori_loop(0, NS, dget, 0)
    plsc.subcore_barrier()

    nwin = (u + WINT - 1) // WINT
    nwin_me = jnp.maximum((nwin - cid + 1) // 2, 0)

    def win(wi2, _):
        base = (wi2 * 2 + cid) * WINT
        pltpu.sync_copy(zeros_hbm, accb)

        def grp(k, _):
            b16 = k * L
            slv = _vload(clslot, b16)
            escv = _vload(clesc, b16)
            gatev = _vload(clgate, b16)
            srcv = jnp.clip(_vload(clsrc, b16), 0, n - 1)
            inl = (b16 + _iota()) < cnt
            inwin = inl & (slv >= base) & (slv < base + WINT)
            idxb[...] = srcv
            cpg = pltpu.make_async_copy(feat.at[idxb], rows, sem)
            cpg.start()
            cpg.wait()
            dv = plsc.load_gather(denb, [jnp.clip(slv, 0, SP - 1)])
            w0 = jnp.where(inwin, escv / (dv + 1e-15), 0.0)
            w1 = w0 * gatev
            rloc = jnp.where(inwin, slv - base, 0)
            for j in range(L):
                w0j = w0[j]
                w1j = w1[j]
                o = rloc[j] * d2
                for t in range(ncw):
                    r = rows[j, pl.ds(t * L, L)]
                    plsc.addupdate(accb.at[pl.ds(o + t * L, L)], r * w0j)
                    plsc.addupdate(accb.at[pl.ds(o + d + t * L, L)], r * w1j)
            return 0

        lax.fori_loop(0, nb, grp, 0)
        pltpu.sync_copy(accb, spacc.at[sid])
        plsc.subcore_barrier()

        r0 = sid * rpt * d2
        for i in range(rpt * d2 // L):
            acc4[pl.ds(i * L, L)] = z

        def redp(pt, _):
            pltpu.sync_copy(spacc.at[pt, pl.ds(r0, rpt * d2)], tmps)
            for i in range(rpt * d2 // L):
                acc4[pl.ds(i * L, L)] = (acc4[pl.ds(i * L, L)]
                                         + tmps[pl.ds(i * L, L)])
            return 0

        lax.fori_loop(0, NS, redp, 0)

        for r in range(rpt):
            grow = base + sid * rpt + r

            @pl.when(grow < u)
            def _():
                pltpu.sync_copy(acc4.at[pl.ds(r * d2, d2)],
                                accp.at[pl.ds(grow * d2, d2)])

        plsc.subcore_barrier()
        return 0

    lax.fori_loop(0, nwin_me, win, 0)


def _k6(d, acc_any, w1_ref, fm_ref, u_ref, h0_any, h1_any,
        abuf, obuf0, obuf1, sem1, sem3, sem4):
    u = u_ref[0, 0]
    bm = 128
    sigfm = jax.nn.sigmoid(fm_ref[...])

    def blk(i, _):
        r0 = i * bm
        cpa = pltpu.make_async_copy(acc_any.at[pl.ds(r0, bm)], abuf, sem1)
        cpa.start()
        cpa.wait()
        acc = abuf[...]
        a0 = acc[:, :d]
        a1 = acc[:, d:] * sigfm
        w1 = w1_ref[...]
        obuf0[...] = jnp.maximum(jnp.dot(a0, w1, preferred_element_type=F32), 0.0)
        obuf1[...] = jnp.maximum(jnp.dot(a1, w1, preferred_element_type=F32), 0.0)
        cpo0 = pltpu.make_async_copy(obuf0, h0_any.at[pl.ds(r0, bm)], sem3)
        cpo1 = pltpu.make_async_copy(obuf1, h1_any.at[pl.ds(r0, bm)], sem4)
        cpo0.start()
        cpo1.start()
        cpo0.wait()
        cpo1.wait()
        return 0

    lax.fori_loop(0, (u + bm - 1) // bm, blk, 0)


def _k7(n, np_, d, h0_hbm, h1_hbm, cnt1f, src1, tag1, esc1, gate1, denxyf,
        nmap_hbm, bpart,
        nmapb, clsrc, cltag, clesc, clgate, cb, dxyb, bacc, rows0, rows1,
        idxb, sem, sem2):
    ncw = d // L
    w = _wid()
    t = jax.lax.rem(w, NS)
    par = w // NS
    ch = clsrc.shape[0] - L
    off = t * ch
    pltpu.sync_copy(nmap_hbm, nmapb.at[pl.ds(0, np_)])
    pltpu.sync_copy(src1.at[pl.ds(off, ch)], clsrc.at[pl.ds(0, ch)])
    pltpu.sync_copy(tag1.at[pl.ds(off, ch)], cltag.at[pl.ds(0, ch)])
    pltpu.sync_copy(esc1.at[pl.ds(off, ch)], clesc.at[pl.ds(0, ch)])
    pltpu.sync_copy(gate1.at[pl.ds(off, ch)], clgate.at[pl.ds(0, ch)])
    pltpu.sync_copy(cnt1f, cb.at[pl.ds(0, NS * L)])
    pltpu.sync_copy(denxyf, dxyb.at[pl.ds(0, NS * L)])
    cnt = _sget(cb, t * L)

    v = jnp.zeros((L,), F32)
    for tt in range(NS):
        v = v + dxyb[pl.ds(tt * L, L)]
    denx = v[0]
    deny = v[1]

    z = jnp.zeros((L,), F32)
    for i in range(4 * d // L):
        bacc[pl.ds(i * L, L)] = z

    def grp(k2, _):
        b16 = (k2 * 2 + par) * L
        srcv = jnp.clip(_vload(clsrc, b16), 0, n - 1)
        tagv = _vload(cltag, b16)
        escv = _vload(clesc, b16)
        gatev = _vload(clgate, b16)
        inl = (b16 + _iota()) < cnt
        slv = plsc.load_gather(nmapb, [srcv])
        idxb[...] = jnp.where(inl, jnp.clip(slv, 0, SP - 1), 0)
        cp0 = pltpu.make_async_copy(h0_hbm.at[idxb], rows0, sem)
        cp1 = pltpu.make_async_copy(h1_hbm.at[idxb], rows1, sem2)
        cp0.start()
        cp1.start()
        cp0.wait()
        cp1.wait()
        denl = jnp.where(tagv == 0, denx, deny)
        w0 = jnp.where(inl, escv / (denl + 1e-15), 0.0)
        w1 = w0 * gatev
        for j in range(L):
            tj = tagv[j]
            w0j = w0[j]
            w1j = w1[j]
            o0 = jnp.clip(tj, 0, 1) * d
            for tc in range(ncw):
                plsc.addupdate(bacc.at[pl.ds(o0 + tc * L, L)],
                               rows0[j, pl.ds(tc * L, L)] * w0j)
                plsc.addupdate(bacc.at[pl.ds(2 * d + o0 + tc * L, L)],
                               rows1[j, pl.ds(tc * L, L)] * w1j)
        return 0

    nbt = (cnt + L - 1) // L
    nbme = jnp.maximum((nbt - par + 1) // 2, 0)
    lax.fori_loop(0, nbme, grp, 0)
    pltpu.sync_copy(bacc.at[pl.ds(0, 4 * d)], bpart.at[pl.ds(w * 4 * d, 4 * d)])


def _k8(e, d, bp_ref, w2_ref, em_ref, fm_ref, xy_ref, out_ref):
    b = jnp.sum(bp_ref[...], axis=0)
    logit = jnp.dot(b, w2_ref[...], preferred_element_type=F32)
    neq = jnp.minimum(jnp.abs(xy_ref[0, 0] - xy_ref[0, 1]), 1).astype(F32)
    l0x = logit[0]
    l0y = logit[1] * neq + logit[0] * (1.0 - neq)
    l1x = logit[2]
    l1y = logit[3] * neq + logit[2] * (1.0 - neq)
    pred = jnp.sum(l0x * l0y)
    lp = jnp.sum(l1x * l1y)

    eps = 1e-15
    em = jax.nn.sigmoid(em_ref[...])
    s_em = jnp.sum(em)
    ent = jnp.sum(-em * jnp.log(em + eps) - (1.0 - em) * jnp.log(1.0 - em + eps))
    fm = jax.nn.sigmoid(fm_ref[...])
    m_fm = jnp.sum(fm) / float(d)
    ent2 = jnp.sum(-fm * jnp.log(fm + eps)
                   - (1.0 - fm) * jnp.log(1.0 - fm + eps)) / float(d)

    loss = (lp - pred) + 0.005 * s_em + ent / float(e) + 1.0 * m_fm + 0.1 * ent2
    out_ref[...] = jnp.reshape(loss, (1, 1))


def kernel(feat, edge_index, feat_mask, edge_mask, W1, W2, x, y):
    n, d = feat.shape
    e = edge_mask.shape[0]
    ch16 = -((-e) // (NS * 128)) * 128
    ep = NS * ch16
    pad = ep - e
    np_ = -((-n) // L) * L
    d2 = 2 * d

    src = edge_index[0]
    dst = edge_index[1]
    srcp = jnp.concatenate([src, jnp.zeros((pad,), I32)])
    dstp = jnp.concatenate([dst, jnp.full((pad,), -1, I32)])
    emp = jnp.concatenate([edge_mask, jnp.full((pad,), -1.0, F32)])
    xi = jnp.asarray(x, I32)
    yi = jnp.asarray(y, I32)
    xy16 = jnp.where(lax.iota(I32, L) == 1, yi, xi)
    zeros_hbm = jnp.zeros((WINT * d2,), F32)

    f32s = jax.ShapeDtypeStruct
    sdma = pltpu.SemaphoreType.DMA

    nsub = 4
    subc = ch16 // nsub
    cnt1f, src1, tag1, esc1, gate1, denxyf, nmap_hbm, u16 = _sc_kernel(
        functools.partial(_ka, n, np_, d, ch16, nsub),
        out_type=[
            f32s((NS * L,), I32), f32s((NS * ch16,), I32),
            f32s((NS * ch16,), I32), f32s((NS * ch16,), F32),
            f32s((NS * ch16,), F32), f32s((NS * L,), F32),
            f32s((np_,), I32), f32s((L,), I32),
        ],
        scratch_types=[
            pltpu.VMEM((subc + L,), I32), pltpu.VMEM((subc + L,), I32),
            pltpu.VMEM((subc + L,), F32),
            pltpu.VMEM((ch16 + L,), I32), pltpu.VMEM((ch16 + L,), I32),
            pltpu.VMEM((ch16 + L,), F32), pltpu.VMEM((ch16 + L,), F32),
            pltpu.VMEM((np_ + L,), I32),
            pltpu.VMEM((L, d), F32), pltpu.VMEM((L, d), F32),
            pltpu.VMEM((L,), I32), pltpu.VMEM((L,), F32),
            pltpu.VMEM_SHARED((NS, ch16 + L), I32),
            sdma,
        ],
    )(feat, srcp, dstp, emp, xy16)

    rpt = WINT // NS
    stripe = SP // NS
    (accp,) = _sc_kernel(
        functools.partial(_kb, n, np_, d, ch16, nsub),
        out_type=[f32s((SP * d2,), F32)],
        scratch_types=[
            pltpu.VMEM((subc + L,), I32), pltpu.VMEM((subc + L,), I32),
            pltpu.VMEM((subc + L,), F32), pltpu.VMEM((np_ + L,), I32),
            pltpu.VMEM((ch16 + L,), I32), pltpu.VMEM((ch16 + L,), I32),
            pltpu.VMEM((ch16 + L,), I32), pltpu.VMEM((ch16 + L,), F32),
            pltpu.VMEM((ch16 + L,), F32), pltpu.VMEM((SP + L,), F32),
            pltpu.VMEM((L, d), F32), pltpu.VMEM((L, d), F32),
            pltpu.VMEM((WINT * d2,), F32), pltpu.VMEM((rpt * d2,), F32),
            pltpu.VMEM((rpt * d2,), F32),
            pltpu.VMEM((stripe + L,), F32), pltpu.VMEM((stripe + L,), F32),
            pltpu.VMEM((L,), I32), pltpu.VMEM((L,), I32),
            pltpu.VMEM_SHARED((NS, WINT * d2), F32),
            sdma, sdma,
        ],
    )(feat, srcp, dstp, emp, nmap_hbm, u16, zeros_hbm)

    u2d = u16[:1].reshape(1, 1)
    acc2d = accp.reshape(SP, d2)
    h0, h1 = pl.pallas_call(
        functools.partial(_k6, d),
        out_shape=[f32s((SP, d), F32), f32s((SP, d), F32)],
        in_specs=[
            pl.BlockSpec(memory_space=pltpu.MemorySpace.HBM),
            pl.BlockSpec(memory_space=pltpu.MemorySpace.VMEM),
            pl.BlockSpec(memory_space=pltpu.MemorySpace.VMEM),
            pl.BlockSpec(memory_space=pltpu.MemorySpace.SMEM),
        ],
        out_specs=[pl.BlockSpec(memory_space=pltpu.MemorySpace.HBM)] * 2,
        scratch_shapes=[
            pltpu.VMEM((128, d2), F32),
            pltpu.VMEM((128, d), F32), pltpu.VMEM((128, d), F32),
            sdma, sdma, sdma,
        ],
    )(acc2d, W1, feat_mask, u2d)

    (bpart,) = _sc_kernel(
        functools.partial(_k7, n, np_, d),
        out_type=[f32s((NW * 4 * d,), F32)],
        scratch_types=[
            pltpu.VMEM((np_ + L,), I32),
            pltpu.VMEM((ch16 + L,), I32), pltpu.VMEM((ch16 + L,), I32),
            pltpu.VMEM((ch16 + L,), F32), pltpu.VMEM((ch16 + L,), F32),
            pltpu.VMEM((NS * L + L,), I32), pltpu.VMEM((NS * L + L,), F32),
            pltpu.VMEM((4 * d,), F32),
            pltpu.VMEM((L, d), F32), pltpu.VMEM((L, d), F32),
            pltpu.VMEM((L,), I32),
            sdma, sdma,
        ],
    )(h0, h1, cnt1f, src1, tag1, esc1, gate1, denxyf, nmap_hbm)

    em2d = edge_mask.reshape(e // 128, 128)
    xy2d = jnp.stack([xi, yi]).reshape(1, 2)
    bp3d = bpart.reshape(NW, 4, d)
    out = pl.pallas_call(
        functools.partial(_k8, e, d),
        out_shape=f32s((1, 1), F32),
        in_specs=[
            pl.BlockSpec(memory_space=pltpu.MemorySpace.VMEM),
            pl.BlockSpec(memory_space=pltpu.MemorySpace.VMEM),
            pl.BlockSpec(memory_space=pltpu.MemorySpace.VMEM),
            pl.BlockSpec(memory_space=pltpu.MemorySpace.VMEM),
            pl.BlockSpec(memory_space=pltpu.MemorySpace.SMEM),
        ],
    )(bp3d, W2, em2d, feat_mask, xy2d)

    return out[0, 0]
